# baseline (speedup 1.0000x reference)
import functools

import jax
import jax.numpy as jnp
from jax import lax
from jax.experimental import pallas as pl
from jax.experimental.pallas import tpu as pltpu
from jax.experimental.pallas import tpu_sc as plsc

_V = 1000000
_B = 16384
_F = 26
_D = 32
_NW = 32
_RPW = _B // _NW
_CHUNK = 64
_NCHUNK = _RPW // _CHUNK
_IDX_PER_CHUNK = _CHUNK * _F
_G = 128
_GPC = _IDX_PER_CHUNK // _G
_GPW = _RPW * _F // _G

_SLANES = 512
_NSTRIP = _V // _SLANES
_TAIL = _V - _NSTRIP * _SLANES
_KMAX = -(-(_NSTRIP + 1) // _NW)


_KPIPE = (_NSTRIP - 1) // _NW


def _repack_body(embt_hbm, tail_hbm, flat_hbm,
                 in0_v, in1_v, in2_v, tail_v, out0_v, out1_v, out2_v,
                 sem_in, sem_out):
    wid = lax.axis_index("s") * 2 + lax.axis_index("c")
    lane = lax.broadcasted_iota(jnp.int32, (16,), 0)
    perm_d = [(lane + dd) & 31 for dd in range(32)]
    inb = (in0_v, in1_v, in2_v)
    outb = (out0_v, out1_v, out2_v)

    def transpose_groups(src_v, dst_v, ngroups):
        def g_body(g, _):
            idx_l = lane + g * 16
            dbase = lane * 32 + g * 512
            for dd in range(32):
                v = plsc.load_gather(src_v, [perm_d[dd], idx_l])
                plsc.store_scatter(dst_v, [dbase + perm_d[dd]], v)
            return 0

        lax.fori_loop(0, ngroups, g_body, 0)

    def in_start(k, b):
        s = k * _NW + wid
        pltpu.async_copy(
            embt_hbm.at[:, pl.ds(s * _SLANES, _SLANES)], inb[b], sem_in)

    def in_wait(b):
        pltpu.make_async_copy(
            embt_hbm.at[:, pl.ds(0, _SLANES)], inb[b], sem_in).wait()

    def out_start(k, b):
        s = k * _NW + wid
        pltpu.async_copy(
            outb[b], flat_hbm.at[pl.ds(s * (_SLANES * _D), _SLANES * _D)],
            sem_out)

    def out_wait(b):
        pltpu.make_async_copy(
            outb[b], flat_hbm.at[pl.ds(0, _SLANES * _D)], sem_out).wait()

    in_start(0, 0)
    in_start(1, 1)

    def body(j, _):
        for t in range(3):
            k = j * 3 + t
            in_wait(t)

            @pl.when(k + 2 <= _KPIPE - 1)
            def _():
                in_start(k + 2, (t + 2) % 3)

            @pl.when(k >= 3)
            def _():
                out_wait(t)

            transpose_groups(inb[t], outb[t], _SLANES // 16)
            out_start(k, t)
        return 0

    lax.fori_loop(0, _KPIPE // 3, body, 0)
    in_wait(0)
    out_wait(0)
    transpose_groups(inb[0], outb[0], _SLANES // 16)
    out_start(_KPIPE - 1, 0)
    out_wait(1)
    out_wait(2)
    out_wait(0)

    @pl.when(wid == 0)
    def _():
        s = _NSTRIP - 1
        pltpu.sync_copy(embt_hbm.at[:, pl.ds(s * _SLANES, _SLANES)], inb[0])
        transpose_groups(inb[0], outb[0], _SLANES // 16)
        pltpu.sync_copy(
            outb[0], flat_hbm.at[pl.ds(s * (_SLANES * _D), _SLANES * _D)])

    @pl.when(wid == 1)
    def _():
        pltpu.sync_copy(tail_hbm, tail_v)
        transpose_groups(tail_v, outb[1], _TAIL // 16)
        pltpu.sync_copy(
            outb[1].at[pl.ds(0, _TAIL * _D)],
            flat_hbm.at[pl.ds(_NSTRIP * _SLANES * _D, _TAIL * _D)])


def _fm_body(x_hbm, lw_hbm, emb_hbm, out_hbm,
             idx_v, rows_v, lin_v, out_v, sem):
    wid = lax.axis_index("s") * 2 + lax.axis_index("c")

    pltpu.sync_copy(x_hbm.at[wid], idx_v)

    lane = lax.broadcasted_iota(jnp.int32, (16,), 0)
    lmask = jnp.where(lane < _F - 16, 1.0, 0.0)

    def chunk_body(c, _):
        copies = []
        for j in range(_GPC):
            idx_row = idx_v.at[c * _GPC + j]
            copies.append(pltpu.async_copy(
                emb_hbm.at[idx_row], rows_v.at[pl.ds(j * _G, _G)], sem))
            copies.append(pltpu.async_copy(
                lw_hbm.at[idx_row], lin_v.at[pl.ds(j * _G, _G)], sem))
        for cp in copies:
            cp.wait()

        def grp_body(g, _):
            def row_body(i, acc):
                base = (g * 16 + i) * _F
                v0 = rows_v[base, pl.ds(0, 16)]
                v1 = rows_v[base, pl.ds(16, 16)]
                s0, q0 = v0, v0 * v0
                s1, q1 = v1, v1 * v1
                for f in range(1, _F):
                    v0 = rows_v[base + f, pl.ds(0, 16)]
                    v1 = rows_v[base + f, pl.ds(16, 16)]
                    s0 = s0 + v0
                    q0 = q0 + v0 * v0
                    s1 = s1 + v1
                    q1 = q1 + v1 * v1
                inter = (s0 * s0 - q0) + (s1 * s1 - q1)
                l0 = lin_v[pl.ds(base, 16)]
                l1 = lin_v[pl.ds(base + 16, 16)]
                t = inter * 0.5 + l0 + l1 * lmask
                return jnp.where(lane == i, jnp.sum(t), acc)

            acc = lax.fori_loop(0, 16, row_body,
                                jnp.zeros((16,), jnp.float32))
            out_v[pl.ds(c * _CHUNK + g * 16, 16)] = acc
            return 0

        lax.fori_loop(0, _CHUNK // 16, grp_body, 0)
        return 0

    lax.fori_loop(0, _NCHUNK, chunk_body, 0)

    pltpu.sync_copy(out_v, out_hbm.at[pl.ds(wid * _RPW, _RPW)])


@jax.jit
def _fm(x_grp, lw_flat, emb_t, emb_tail):
    mesh = plsc.VectorSubcoreMesh(core_axis_name="c", subcore_axis_name="s")
    emb_flat = pl.kernel(
        _repack_body,
        out_type=jax.ShapeDtypeStruct((_V * _D,), jnp.float32),
        mesh=mesh,
        compiler_params=pltpu.CompilerParams(
            needs_layout_passes=False, use_tc_tiling_on_sc=True),
        scratch_types=[
            pltpu.VMEM((32, _SLANES), jnp.float32),
            pltpu.VMEM((32, _SLANES), jnp.float32),
            pltpu.VMEM((32, _SLANES), jnp.float32),
            pltpu.VMEM((32, _TAIL), jnp.float32),
            pltpu.VMEM((_SLANES * _D,), jnp.float32),
            pltpu.VMEM((_SLANES * _D,), jnp.float32),
            pltpu.VMEM((_SLANES * _D,), jnp.float32),
            pltpu.SemaphoreType.DMA,
            pltpu.SemaphoreType.DMA,
        ],
    )(emb_t, emb_tail)

    return pl.kernel(
        _fm_body,
        out_type=jax.ShapeDtypeStruct((_B,), jnp.float32),
        mesh=mesh,
        compiler_params=pltpu.CompilerParams(
            needs_layout_passes=False, use_tc_tiling_on_sc=False),
        scratch_types=[
            pltpu.VMEM((_GPW, _G), jnp.int32),
            pltpu.VMEM((_IDX_PER_CHUNK, _D), jnp.float32),
            pltpu.VMEM((_IDX_PER_CHUNK + 16,), jnp.float32),
            pltpu.VMEM((_RPW,), jnp.float32),
            pltpu.SemaphoreType.DMA,
        ],
    )(x_grp, lw_flat, emb_flat.reshape(_V, _D))


def kernel(x, global_bias, linear_weights, interaction_factors):
    x_grp = x.astype(jnp.int32).reshape(_NW, _GPW, _G)
    lw_flat = linear_weights.reshape(-1)
    emb_t = interaction_factors.T
    out = _fm(x_grp, lw_flat, emb_t, emb_t[:, _NSTRIP * _SLANES:])
    return out + global_bias[0]


# speedup vs baseline: 1.0405x; 1.0405x over previous
import functools

import jax
import jax.numpy as jnp
from jax import lax
from jax.experimental import pallas as pl
from jax.experimental.pallas import tpu as pltpu
from jax.experimental.pallas import tpu_sc as plsc

_V = 1000000
_B = 16384
_F = 26
_D = 32
_NW = 32
_RPW = _B // _NW
_CHUNK = 64
_NCHUNK = _RPW // _CHUNK
_IDX_PER_CHUNK = _CHUNK * _F
_G = 128
_GPC = _IDX_PER_CHUNK // _G
_GPW = _RPW * _F // _G

_SLANES = 512
_NSTRIP = _V // _SLANES
_TAIL = _V - _NSTRIP * _SLANES
_KMAX = -(-(_NSTRIP + 1) // _NW)


_KPIPE = (_NSTRIP - 1) // _NW


def _repack_body(embt_hbm, tail_hbm, flat_hbm,
                 in0_v, in1_v, in2_v, tail_v, out0_v, out1_v, out2_v,
                 sem_in, sem_out):
    wid = lax.axis_index("s") * 2 + lax.axis_index("c")
    lane = lax.broadcasted_iota(jnp.int32, (16,), 0)
    perm_d = [(lane + dd) & 31 for dd in range(32)]
    inb = (in0_v, in1_v, in2_v)
    outb = (out0_v, out1_v, out2_v)

    def transpose_groups(src_v, dst_v, ngroups):
        def g_body(g, _):
            idx_l = lane + g * 16
            dbase = lane * 32 + g * 512
            for dd in range(32):
                v = plsc.load_gather(src_v, [perm_d[dd], idx_l])
                plsc.store_scatter(dst_v, [dbase + perm_d[dd]], v)
            return 0

        lax.fori_loop(0, ngroups, g_body, 0)

    def in_start(k, b):
        s = k * _NW + wid
        pltpu.async_copy(
            embt_hbm.at[:, pl.ds(s * _SLANES, _SLANES)], inb[b], sem_in)

    def in_wait(b):
        pltpu.make_async_copy(
            embt_hbm.at[:, pl.ds(0, _SLANES)], inb[b], sem_in).wait()

    def out_start(k, b):
        s = k * _NW + wid
        pltpu.async_copy(
            outb[b], flat_hbm.at[pl.ds(s * (_SLANES * _D), _SLANES * _D)],
            sem_out)

    def out_wait(b):
        pltpu.make_async_copy(
            outb[b], flat_hbm.at[pl.ds(0, _SLANES * _D)], sem_out).wait()

    in_start(0, 0)
    in_start(1, 1)

    def body(j, _):
        for t in range(3):
            k = j * 3 + t
            in_wait(t)

            @pl.when(k + 2 <= _KPIPE - 1)
            def _():
                in_start(k + 2, (t + 2) % 3)

            @pl.when(k >= 3)
            def _():
                out_wait(t)

            transpose_groups(inb[t], outb[t], _SLANES // 16)
            out_start(k, t)
        return 0

    lax.fori_loop(0, _KPIPE // 3, body, 0)
    in_wait(0)
    out_wait(0)
    transpose_groups(inb[0], outb[0], _SLANES // 16)
    out_start(_KPIPE - 1, 0)
    out_wait(1)
    out_wait(2)
    out_wait(0)

    @pl.when(wid == 0)
    def _():
        s = _NSTRIP - 1
        pltpu.sync_copy(embt_hbm.at[:, pl.ds(s * _SLANES, _SLANES)], inb[0])
        transpose_groups(inb[0], outb[0], _SLANES // 16)
        pltpu.sync_copy(
            outb[0], flat_hbm.at[pl.ds(s * (_SLANES * _D), _SLANES * _D)])

    @pl.when(wid == 1)
    def _():
        pltpu.sync_copy(tail_hbm, tail_v)
        transpose_groups(tail_v, outb[1], _TAIL // 16)
        pltpu.sync_copy(
            outb[1].at[pl.ds(0, _TAIL * _D)],
            flat_hbm.at[pl.ds(_NSTRIP * _SLANES * _D, _TAIL * _D)])


def _fm_body(x_hbm, lw_hbm, emb_hbm, out_hbm,
             idx_v, rows0_v, rows1_v, lin0_v, lin1_v, out_v, sem):
    wid = lax.axis_index("s") * 2 + lax.axis_index("c")
    rowsb = (rows0_v, rows1_v)
    linb = (lin0_v, lin1_v)

    pltpu.sync_copy(x_hbm.at[wid], idx_v)

    lane = lax.broadcasted_iota(jnp.int32, (16,), 0)
    lmask = jnp.where(lane < _F - 16, 1.0, 0.0)

    def fire(c, b):
        for j in range(_GPC):
            idx_row = idx_v.at[c * _GPC + j]
            pltpu.async_copy(
                emb_hbm.at[idx_row], rowsb[b].at[pl.ds(j * _G, _G)], sem)
            pltpu.async_copy(
                lw_hbm.at[idx_row], linb[b].at[pl.ds(j * _G, _G)], sem)

    def drain(b):
        for j in range(_GPC):
            pltpu.make_async_copy(
                emb_hbm.at[idx_v.at[0]],
                rowsb[b].at[pl.ds(j * _G, _G)], sem).wait()
            pltpu.make_async_copy(
                lw_hbm.at[idx_v.at[0]],
                linb[b].at[pl.ds(j * _G, _G)], sem).wait()

    def compute(c, b):
        rows_v = rowsb[b]
        lin_v = linb[b]

        def grp_body(g, _):
            def row_body(i, acc):
                base = (g * 16 + i) * _F
                v0 = rows_v[base, pl.ds(0, 16)]
                v1 = rows_v[base, pl.ds(16, 16)]
                s0, q0 = v0, v0 * v0
                s1, q1 = v1, v1 * v1
                for f in range(1, _F):
                    v0 = rows_v[base + f, pl.ds(0, 16)]
                    v1 = rows_v[base + f, pl.ds(16, 16)]
                    s0 = s0 + v0
                    q0 = q0 + v0 * v0
                    s1 = s1 + v1
                    q1 = q1 + v1 * v1
                inter = (s0 * s0 - q0) + (s1 * s1 - q1)
                l0 = lin_v[pl.ds(base, 16)]
                l1 = lin_v[pl.ds(base + 16, 16)]
                t = inter * 0.5 + l0 + l1 * lmask
                return jnp.where(lane == i, jnp.sum(t), acc)

            acc = lax.fori_loop(0, 16, row_body,
                                jnp.zeros((16,), jnp.float32))
            out_v[pl.ds(c * _CHUNK + g * 16, 16)] = acc
            return 0

        lax.fori_loop(0, _CHUNK // 16, grp_body, 0)

    fire(0, 0)

    def pair_body(j, _):
        c = j * 2
        drain(0)
        fire(c + 1, 1)
        compute(c, 0)
        drain(1)

        @pl.when(c + 2 < _NCHUNK)
        def _():
            fire(c + 2, 0)

        compute(c + 1, 1)
        return 0

    lax.fori_loop(0, _NCHUNK // 2, pair_body, 0)

    pltpu.sync_copy(out_v, out_hbm.at[pl.ds(wid * _RPW, _RPW)])


@jax.jit
def _fm(x_grp, lw_flat, emb_t, emb_tail):
    mesh = plsc.VectorSubcoreMesh(core_axis_name="c", subcore_axis_name="s")
    emb_flat = pl.kernel(
        _repack_body,
        out_type=jax.ShapeDtypeStruct((_V * _D,), jnp.float32),
        mesh=mesh,
        compiler_params=pltpu.CompilerParams(
            needs_layout_passes=False, use_tc_tiling_on_sc=True),
        scratch_types=[
            pltpu.VMEM((32, _SLANES), jnp.float32),
            pltpu.VMEM((32, _SLANES), jnp.float32),
            pltpu.VMEM((32, _SLANES), jnp.float32),
            pltpu.VMEM((32, _TAIL), jnp.float32),
            pltpu.VMEM((_SLANES * _D,), jnp.float32),
            pltpu.VMEM((_SLANES * _D,), jnp.float32),
            pltpu.VMEM((_SLANES * _D,), jnp.float32),
            pltpu.SemaphoreType.DMA,
            pltpu.SemaphoreType.DMA,
        ],
    )(emb_t, emb_tail)

    return pl.kernel(
        _fm_body,
        out_type=jax.ShapeDtypeStruct((_B,), jnp.float32),
        mesh=mesh,
        compiler_params=pltpu.CompilerParams(
            needs_layout_passes=False, use_tc_tiling_on_sc=False),
        scratch_types=[
            pltpu.VMEM((_GPW, _G), jnp.int32),
            pltpu.VMEM((_IDX_PER_CHUNK, _D), jnp.float32),
            pltpu.VMEM((_IDX_PER_CHUNK, _D), jnp.float32),
            pltpu.VMEM((_IDX_PER_CHUNK + 16,), jnp.float32),
            pltpu.VMEM((_IDX_PER_CHUNK + 16,), jnp.float32),
            pltpu.VMEM((_RPW,), jnp.float32),
            pltpu.SemaphoreType.DMA,
        ],
    )(x_grp, lw_flat, emb_flat.reshape(_V, _D))


def kernel(x, global_bias, linear_weights, interaction_factors):
    x_grp = x.astype(jnp.int32).reshape(_NW, _GPW, _G)
    lw_flat = linear_weights.reshape(-1)
    emb_t = interaction_factors.T
    out = _fm(x_grp, lw_flat, emb_t, emb_t[:, _NSTRIP * _SLANES:])
    return out + global_bias[0]
